# baseline (speedup 1.0000x reference)
import jax
import jax.numpy as jnp
from jax.experimental import pallas as pl
from jax.experimental.pallas import tpu as pltpu

IN_DIM = 1024
H_ROUTER = 256
E = 8
TOP_K = 2
HID = 1024
NUM_CLASSES = 512
TOK = 4096

TBLK = 512


def _router_kernel(x_ref, w1_ref, b1_ref, w2_ref, b2_ref, p_ref, w_ref):
    x = x_ref[...]
    h = jnp.maximum(jnp.dot(x, w1_ref[...], preferred_element_type=jnp.float32)
                    + b1_ref[...], 0.0)
    g = jnp.dot(h, w2_ref[...], preferred_element_type=jnp.float32) + b2_ref[...]
    m = jnp.max(g, axis=1, keepdims=True)
    ex = jnp.exp(g - m)
    p = ex / jnp.sum(ex, axis=1, keepdims=True)
    p_ref[...] = p

    lanes = jax.lax.broadcasted_iota(jnp.int32, p.shape, 1)
    m1 = jnp.max(p, axis=1, keepdims=True)
    i1 = jnp.min(jnp.where(p == m1, lanes, E), axis=1, keepdims=True)
    masked = jnp.where(lanes == i1, -jnp.inf, p)
    m2 = jnp.max(masked, axis=1, keepdims=True)
    i2 = jnp.min(jnp.where(masked == m2, lanes, E), axis=1, keepdims=True)
    w_ref[...] = jnp.where(lanes == i1, m1, 0.0) + jnp.where(lanes == i2, m2, 0.0)


def _expert_kernel(x_ref, w1_ref, b1_ref, w2_ref, b2_ref, wt_ref, out_ref):
    e = pl.program_id(1)

    @pl.when(e == 0)
    def _():
        out_ref[...] = jnp.zeros_like(out_ref)

    x = x_ref[...]
    h = jnp.maximum(jnp.dot(x, w1_ref[0], preferred_element_type=jnp.float32)
                    + b1_ref[0], 0.0)
    y = jnp.dot(h, w2_ref[0], preferred_element_type=jnp.float32) + b2_ref[0]
    w = wt_ref[...].reshape(TBLK, 1)
    out_ref[...] += y * w


def kernel(x, router_W1, router_b1, router_W2, router_b2,
           exp_W1, exp_b1, exp_W2, exp_b2):
    nt = TOK // TBLK

    router_p, w_dense = pl.pallas_call(
        _router_kernel,
        grid=(nt,),
        in_specs=[
            pl.BlockSpec((TBLK, IN_DIM), lambda t: (t, 0)),
            pl.BlockSpec((IN_DIM, H_ROUTER), lambda t: (0, 0)),
            pl.BlockSpec((1, H_ROUTER), lambda t: (0, 0)),
            pl.BlockSpec((H_ROUTER, E), lambda t: (0, 0)),
            pl.BlockSpec((1, E), lambda t: (0, 0)),
        ],
        out_specs=[
            pl.BlockSpec((TBLK, E), lambda t: (t, 0)),
            pl.BlockSpec((TBLK, E), lambda t: (t, 0)),
        ],
        out_shape=[
            jax.ShapeDtypeStruct((TOK, E), jnp.float32),
            jax.ShapeDtypeStruct((TOK, E), jnp.float32),
        ],
    )(x, router_W1, router_b1.reshape(1, H_ROUTER),
      router_W2, router_b2.reshape(1, E))

    wT = w_dense.T

    out = pl.pallas_call(
        _expert_kernel,
        grid=(nt, E),
        in_specs=[
            pl.BlockSpec((TBLK, IN_DIM), lambda t, e: (t, 0)),
            pl.BlockSpec((1, IN_DIM, HID), lambda t, e: (e, 0, 0)),
            pl.BlockSpec((1, 1, HID), lambda t, e: (e, 0, 0)),
            pl.BlockSpec((1, HID, NUM_CLASSES), lambda t, e: (e, 0, 0)),
            pl.BlockSpec((1, 1, NUM_CLASSES), lambda t, e: (e, 0, 0)),
            pl.BlockSpec((1, 1, TBLK), lambda t, e: (e, 0, t)),
        ],
        out_specs=pl.BlockSpec((TBLK, NUM_CLASSES), lambda t, e: (t, 0)),
        out_shape=jax.ShapeDtypeStruct((TOK, NUM_CLASSES), jnp.float32),
    )(x, exp_W1, exp_b1.reshape(E, 1, HID), exp_W2,
      exp_b2.reshape(E, 1, NUM_CLASSES), wT.reshape(E, 1, TOK))

    out = out * (1.0 / TOP_K)
    lb_loss = jnp.asarray(0.0, jnp.float32)
    return (out, router_p, lb_loss)


# speedup vs baseline: 1.0465x; 1.0465x over previous
import functools

import jax
import jax.numpy as jnp
from jax import lax
from jax.experimental import pallas as pl
from jax.experimental.pallas import tpu as pltpu
from jax.experimental.pallas import tpu_sc as plsc

IN_DIM = 1024
H_ROUTER = 256
E = 8
TOP_K = 2
HID = 1024
NUM_CLASSES = 512
TOK = 4096

TBLK = 256
NPAIR = TOK * TOP_K
BM = 256
NB = NPAIR // BM + E
P = NB * BM

NC = 2
NS = 16
LANE = 16
NW = NC * NS

_SC_MESH = functools.partial(
    plsc.VectorSubcoreMesh, core_axis_name="c", subcore_axis_name="s")
_SC_PARAMS = pltpu.CompilerParams(needs_layout_passes=False)


def _router_kernel(x_ref, w1_ref, b1_ref, w2_ref, b2_ref,
                   p_ref, eidx_ref, ew_ref, hist_ref):
    x = x_ref[...]
    h = jnp.maximum(jnp.dot(x, w1_ref[...], preferred_element_type=jnp.float32)
                    + b1_ref[...], 0.0)
    g = jnp.dot(h, w2_ref[...], preferred_element_type=jnp.float32) + b2_ref[...]
    m = jnp.max(g, axis=1, keepdims=True)
    ex = jnp.exp(g - m)
    p = ex / jnp.sum(ex, axis=1, keepdims=True)
    p_ref[...] = p

    lanes = lax.broadcasted_iota(jnp.int32, p.shape, 1)
    m1 = jnp.max(p, axis=1, keepdims=True)
    i1 = jnp.min(jnp.where(p == m1, lanes, E), axis=1, keepdims=True)
    masked = jnp.where(lanes == i1, -jnp.inf, p)
    m2 = jnp.max(masked, axis=1, keepdims=True)
    i2 = jnp.min(jnp.where(masked == m2, lanes, E), axis=1, keepdims=True)

    col = lax.broadcasted_iota(jnp.int32, (x.shape[0], TOP_K), 1)
    eidx = jnp.where(col == 0, i1, i2)
    eidx_ref[...] = eidx
    ew_ref[...] = jnp.where(col == 0, m1, m2) * (1.0 / TOP_K)

    hlane = lax.broadcasted_iota(jnp.int32, (1, LANE), 1)
    row = jnp.zeros((1, LANE), jnp.int32)
    for e in range(E):
        cnt = jnp.sum(jnp.where(eidx == e, 1, 0))
        row = row + jnp.where(hlane == e, cnt, 0)
    hist_ref[pl.ds(pl.program_id(0), 1), :] = row


def _run_router(x, rW1, rb1, rW2, rb2):
    nt = TOK // TBLK
    return pl.pallas_call(
        _router_kernel,
        grid=(nt,),
        in_specs=[
            pl.BlockSpec((TBLK, IN_DIM), lambda t: (t, 0)),
            pl.BlockSpec((IN_DIM, H_ROUTER), lambda t: (0, 0)),
            pl.BlockSpec((1, H_ROUTER), lambda t: (0, 0)),
            pl.BlockSpec((H_ROUTER, E), lambda t: (0, 0)),
            pl.BlockSpec((1, E), lambda t: (0, 0)),
        ],
        out_specs=[
            pl.BlockSpec((TBLK, E), lambda t: (t, 0)),
            pl.BlockSpec((TBLK, TOP_K), lambda t: (t, 0)),
            pl.BlockSpec((TBLK, TOP_K), lambda t: (t, 0)),
            pl.BlockSpec((TOK // TBLK, LANE), lambda t: (0, 0)),
        ],
        out_shape=[
            jax.ShapeDtypeStruct((TOK, E), jnp.float32),
            jax.ShapeDtypeStruct((TOK, TOP_K), jnp.int32),
            jax.ShapeDtypeStruct((TOK, TOP_K), jnp.float32),
            jax.ShapeDtypeStruct((TOK // TBLK, LANE), jnp.int32),
        ],
    )(x, rW1, rb1.reshape(1, H_ROUTER), rW2, rb2.reshape(1, E))


CPAIR = NPAIR // NS
NBPAD = 48


def _slots_body(eidx_hbm, hist_hbm, pos_hbm, blk_hbm,
                eidx_v, allhist_v, slots_v, blk_v):
    sid = lax.axis_index("s")
    base = sid * CPAIR
    lane = lax.broadcasted_iota(jnp.int32, (LANE,), 0)

    pltpu.sync_copy(eidx_hbm.at[pl.ds(base, CPAIR)], eidx_v)
    pltpu.sync_copy(hist_hbm, allhist_v)

    acc = jnp.zeros((LANE,), jnp.int32)
    tiles_before = jnp.zeros((LANE,), jnp.int32)
    for w in range(NS):
        tiles_before = jnp.where(sid == w, acc, tiles_before)
        acc = acc + allhist_v[w, :]
    totals = acc
    padded = ((totals + (BM - 1)) >> 8) << 8
    cum_incl = plsc.cumsum(padded)
    offsets = cum_incl - padded
    cum_end = offsets + padded
    tile_base = offsets + tiles_before

    def slot_step(v, cnt):
        keys = eidx_v[pl.ds(v * LANE, LANE)]
        slot = jnp.zeros((LANE,), jnp.int32)
        for e in range(E):
            m = keys == e
            mi = jnp.where(m, 1, 0)
            incl = plsc.cumsum(mi)
            cnt_e = jnp.sum(jnp.where(lane == e, cnt, 0))
            slot = jnp.where(m, cnt_e + incl - 1, slot)
            cnt = cnt + jnp.where(lane == e, jnp.sum(mi), 0)
        slots_v[pl.ds(v * LANE, LANE)] = slot
        return cnt

    lax.fori_loop(0, CPAIR // LANE, slot_step, tile_base)

    pltpu.sync_copy(slots_v, pos_hbm.at[pl.ds(base, CPAIR)])

    @pl.when(sid == 0)
    def _():
        ce = [jnp.sum(jnp.where(lane == e, cum_end, 0)) for e in range(E)]
        for g in range(NBPAD // LANE):
            s_vec = (g * LANE + lane) * BM
            cnt = jnp.zeros((LANE,), jnp.int32)
            for e in range(E):
                cnt = cnt + jnp.where(ce[e] <= s_vec, 1, 0)
            blk_v[pl.ds(g * LANE, LANE)] = jnp.minimum(cnt, E - 1)
        pltpu.sync_copy(blk_v, blk_hbm)


def _run_slots(eidx_flat, hist):
    return pl.kernel(
        _slots_body,
        out_type=[
            jax.ShapeDtypeStruct((NPAIR,), jnp.int32),
            jax.ShapeDtypeStruct((NBPAD,), jnp.int32),
        ],
        mesh=_SC_MESH(num_cores=1),
        compiler_params=_SC_PARAMS,
        scratch_types=[
            pltpu.VMEM((CPAIR,), jnp.int32),
            pltpu.VMEM((NS, LANE), jnp.int32),
            pltpu.VMEM((CPAIR,), jnp.int32),
            pltpu.VMEM((NBPAD,), jnp.int32),
        ],
    )(eidx_flat, hist)


TPW = TOK // NW
XCH = 16


def _dispatch_body(pos_hbm, x_hbm, xs_hbm, posv, xr, sem):
    wid = lax.axis_index("s") * NC + lax.axis_index("c")
    pbase = wid * TPW * TOP_K
    tbase = wid * TPW
    lane = lax.broadcasted_iota(jnp.int32, (LANE,), 0)
    pltpu.sync_copy(pos_hbm.at[pl.ds(pbase, TPW * TOP_K)], posv)
    for j in range(TPW // XCH):
        pltpu.sync_copy(x_hbm.at[pl.ds(tbase + j * XCH, XCH)], xr)
        pair0 = (j * XCH + lane) * TOP_K
        idx_even = plsc.load_gather(posv, [pair0])
        idx_odd = plsc.load_gather(posv, [pair0 + 1])
        idx_even = jnp.minimum(jnp.maximum(idx_even, 0), P - 1)
        idx_odd = jnp.minimum(jnp.maximum(idx_odd, 0), P - 1)
        cp_e = pltpu.async_copy(xr, xs_hbm.at[idx_even], sem)
        cp_o = pltpu.async_copy(xr, xs_hbm.at[idx_odd], sem)
        cp_e.wait()
        cp_o.wait()


def _run_dispatch(pos, x):
    return pl.kernel(
        _dispatch_body,
        out_type=jax.ShapeDtypeStruct((P, IN_DIM), jnp.float32),
        mesh=_SC_MESH(),
        compiler_params=_SC_PARAMS,
        scratch_types=[
            pltpu.VMEM((TPW * TOP_K,), jnp.int32),
            pltpu.VMEM((XCH, IN_DIM), jnp.float32),
            pltpu.SemaphoreType.DMA,
        ],
    )(pos, x)


def _mlp_kernel(be_ref, xs_ref, w1_ref, b1_ref, w2_ref, b2_ref, y_ref):
    del be_ref
    h = jnp.maximum(
        jnp.dot(xs_ref[...], w1_ref[0], preferred_element_type=jnp.float32)
        + b1_ref[0], 0.0)
    y_ref[...] = jnp.dot(h, w2_ref[0], preferred_element_type=jnp.float32) + b2_ref[0]


def _run_mlp(blk_expert, xs, eW1, eb1, eW2, eb2):
    grid_spec = pltpu.PrefetchScalarGridSpec(
        num_scalar_prefetch=1,
        grid=(NB,),
        in_specs=[
            pl.BlockSpec((BM, IN_DIM), lambda b, be: (b, 0)),
            pl.BlockSpec((1, IN_DIM, HID), lambda b, be: (be[b], 0, 0)),
            pl.BlockSpec((1, 1, HID), lambda b, be: (be[b], 0, 0)),
            pl.BlockSpec((1, HID, NUM_CLASSES), lambda b, be: (be[b], 0, 0)),
            pl.BlockSpec((1, 1, NUM_CLASSES), lambda b, be: (be[b], 0, 0)),
        ],
        out_specs=pl.BlockSpec((BM, NUM_CLASSES), lambda b, be: (b, 0)),
    )
    return pl.pallas_call(
        _mlp_kernel,
        grid_spec=grid_spec,
        out_shape=jax.ShapeDtypeStruct((P, NUM_CLASSES), jnp.float32),
    )(blk_expert, xs, eW1, eb1.reshape(E, 1, HID), eW2,
      eb2.reshape(E, 1, NUM_CLASSES))


CCH = 32


def _combine_body(pos_hbm, ew_hbm, y_hbm, out_hbm, posv, ewv, idx_v, yv, ov,
                  sem):
    wid = lax.axis_index("s") * NC + lax.axis_index("c")
    pbase = wid * TPW * TOP_K
    tbase = wid * TPW
    lane = lax.broadcasted_iota(jnp.int32, (LANE,), 0)
    pltpu.sync_copy(pos_hbm.at[pl.ds(pbase, TPW * TOP_K)], posv)
    pltpu.sync_copy(ew_hbm.at[pl.ds(pbase, TPW * TOP_K)], ewv)
    for c in range(TPW // CCH):
        for j in range(CCH * TOP_K // LANE):
            v = posv[pl.ds(c * CCH * TOP_K + j * LANE, LANE)]
            idx_v[pl.ds(j * LANE, LANE)] = jnp.minimum(
                jnp.maximum(v, 0), P - 1)
        pltpu.async_copy(y_hbm.at[idx_v], yv, sem).wait()

        def add_step(i, _):
            g0 = (i >> 3) * LANE
            ewg = ewv[pl.ds(c * CCH * TOP_K + g0, LANE)]
            l0 = (2 * i) & (LANE - 1)
            w0 = jnp.sum(jnp.where(lane == l0, ewg, 0.0))
            w1 = jnp.sum(jnp.where(lane == l0 + 1, ewg, 0.0))
            for j in range(NUM_CLASSES // LANE):
                a = yv[2 * i, pl.ds(j * LANE, LANE)]
                b = yv[2 * i + 1, pl.ds(j * LANE, LANE)]
                ov[i, pl.ds(j * LANE, LANE)] = a * w0 + b * w1
            return 0

        lax.fori_loop(0, CCH, add_step, 0)
        pltpu.sync_copy(ov, out_hbm.at[pl.ds(tbase + c * CCH, CCH)])


def _run_combine(pos, ew_flat, y):
    return pl.kernel(
        _combine_body,
        out_type=jax.ShapeDtypeStruct((TOK, NUM_CLASSES), jnp.float32),
        mesh=_SC_MESH(),
        compiler_params=_SC_PARAMS,
        scratch_types=[
            pltpu.VMEM((TPW * TOP_K,), jnp.int32),
            pltpu.VMEM((TPW * TOP_K,), jnp.float32),
            pltpu.VMEM((CCH * TOP_K,), jnp.int32),
            pltpu.VMEM((CCH * TOP_K, NUM_CLASSES), jnp.float32),
            pltpu.VMEM((CCH, NUM_CLASSES), jnp.float32),
            pltpu.SemaphoreType.DMA,
        ],
    )(pos, ew_flat, y)


def kernel(x, router_W1, router_b1, router_W2, router_b2,
           exp_W1, exp_b1, exp_W2, exp_b2):
    router_p, eidx, ew, hist = _run_router(
        x, router_W1, router_b1, router_W2, router_b2)

    pos, blk_expert = _run_slots(eidx.reshape(NPAIR), hist)

    xs = _run_dispatch(pos, x)
    y = _run_mlp(blk_expert, xs, exp_W1, exp_b1, exp_W2, exp_b2)
    out = _run_combine(pos, ew.reshape(NPAIR), y)

    lb_loss = jnp.asarray(0.0, jnp.float32)
    return (out, router_p, lb_loss)


# speedup vs baseline: 1.1257x; 1.0756x over previous
import functools

import jax
import jax.numpy as jnp
from jax import lax
from jax.experimental import pallas as pl
from jax.experimental.pallas import tpu as pltpu
from jax.experimental.pallas import tpu_sc as plsc

IN_DIM = 1024
H_ROUTER = 256
E = 8
TOP_K = 2
HID = 1024
NUM_CLASSES = 512
TOK = 4096

TBLK = 256
NPAIR = TOK * TOP_K
BM = 256
NB = NPAIR // BM + E
P = NB * BM

NC = 2
NS = 16
LANE = 16
NW = NC * NS

_SC_MESH = functools.partial(
    plsc.VectorSubcoreMesh, core_axis_name="c", subcore_axis_name="s")
_SC_PARAMS = pltpu.CompilerParams(needs_layout_passes=False)


def _router_kernel(x_ref, w1_ref, b1_ref, w2_ref, b2_ref,
                   p_ref, eidx_ref, ew_ref, hist_ref):
    x = x_ref[...]
    h = jnp.maximum(jnp.dot(x, w1_ref[...], preferred_element_type=jnp.float32)
                    + b1_ref[...], 0.0)
    g = jnp.dot(h, w2_ref[...], preferred_element_type=jnp.float32) + b2_ref[...]
    m = jnp.max(g, axis=1, keepdims=True)
    ex = jnp.exp(g - m)
    p = ex / jnp.sum(ex, axis=1, keepdims=True)
    p_ref[...] = p

    lanes = lax.broadcasted_iota(jnp.int32, p.shape, 1)
    m1 = jnp.max(p, axis=1, keepdims=True)
    i1 = jnp.min(jnp.where(p == m1, lanes, E), axis=1, keepdims=True)
    masked = jnp.where(lanes == i1, -jnp.inf, p)
    m2 = jnp.max(masked, axis=1, keepdims=True)
    i2 = jnp.min(jnp.where(masked == m2, lanes, E), axis=1, keepdims=True)

    col = lax.broadcasted_iota(jnp.int32, (x.shape[0], TOP_K), 1)
    eidx = jnp.where(col == 0, i1, i2)
    eidx_ref[...] = eidx
    ew_ref[...] = jnp.where(col == 0, m1, m2) * (1.0 / TOP_K)

    hlane = lax.broadcasted_iota(jnp.int32, (1, LANE), 1)
    row = jnp.zeros((1, LANE), jnp.int32)
    for e in range(E):
        cnt = jnp.sum(jnp.where(eidx == e, 1, 0))
        row = row + jnp.where(hlane == e, cnt, 0)
    hist_ref[pl.ds(pl.program_id(0), 1), :] = row


def _run_router(x, rW1, rb1, rW2, rb2):
    nt = TOK // TBLK
    return pl.pallas_call(
        _router_kernel,
        grid=(nt,),
        in_specs=[
            pl.BlockSpec((TBLK, IN_DIM), lambda t: (t, 0)),
            pl.BlockSpec((IN_DIM, H_ROUTER), lambda t: (0, 0)),
            pl.BlockSpec((1, H_ROUTER), lambda t: (0, 0)),
            pl.BlockSpec((H_ROUTER, E), lambda t: (0, 0)),
            pl.BlockSpec((1, E), lambda t: (0, 0)),
        ],
        out_specs=[
            pl.BlockSpec((TBLK, E), lambda t: (t, 0)),
            pl.BlockSpec((TBLK, TOP_K), lambda t: (t, 0)),
            pl.BlockSpec((TBLK, TOP_K), lambda t: (t, 0)),
            pl.BlockSpec((TOK // TBLK, LANE), lambda t: (0, 0)),
        ],
        out_shape=[
            jax.ShapeDtypeStruct((TOK, E), jnp.float32),
            jax.ShapeDtypeStruct((TOK, TOP_K), jnp.int32),
            jax.ShapeDtypeStruct((TOK, TOP_K), jnp.float32),
            jax.ShapeDtypeStruct((TOK // TBLK, LANE), jnp.int32),
        ],
    )(x, rW1, rb1.reshape(1, H_ROUTER), rW2, rb2.reshape(1, E))


CPAIR = NPAIR // NS
NBPAD = 48


def _slots_body(eidx_hbm, hist_hbm, pos_hbm, blk_hbm,
                eidx_v, allhist_v, slots_v, blk_v):
    sid = lax.axis_index("s")
    base = sid * CPAIR
    lane = lax.broadcasted_iota(jnp.int32, (LANE,), 0)

    pltpu.sync_copy(eidx_hbm.at[pl.ds(base, CPAIR)], eidx_v)
    pltpu.sync_copy(hist_hbm, allhist_v)

    acc = jnp.zeros((LANE,), jnp.int32)
    tiles_before = jnp.zeros((LANE,), jnp.int32)
    for w in range(NS):
        tiles_before = jnp.where(sid == w, acc, tiles_before)
        acc = acc + allhist_v[w, :]
    totals = acc
    padded = ((totals + (BM - 1)) >> 8) << 8
    cum_incl = plsc.cumsum(padded)
    offsets = cum_incl - padded
    cum_end = offsets + padded
    tile_base = offsets + tiles_before

    def slot_step(v, cnt):
        keys = eidx_v[pl.ds(v * LANE, LANE)]
        slot = jnp.zeros((LANE,), jnp.int32)
        for e in range(E):
            m = keys == e
            mi = jnp.where(m, 1, 0)
            incl = plsc.cumsum(mi)
            cnt_e = jnp.sum(jnp.where(lane == e, cnt, 0))
            slot = jnp.where(m, cnt_e + incl - 1, slot)
            cnt = cnt + jnp.where(lane == e, jnp.sum(mi), 0)
        slots_v[pl.ds(v * LANE, LANE)] = slot
        return cnt

    lax.fori_loop(0, CPAIR // LANE, slot_step, tile_base)

    pltpu.sync_copy(slots_v, pos_hbm.at[pl.ds(base, CPAIR)])

    @pl.when(sid == 0)
    def _():
        ce = [jnp.sum(jnp.where(lane == e, cum_end, 0)) for e in range(E)]
        for g in range(NBPAD // LANE):
            s_vec = (g * LANE + lane) * BM
            cnt = jnp.zeros((LANE,), jnp.int32)
            for e in range(E):
                cnt = cnt + jnp.where(ce[e] <= s_vec, 1, 0)
            blk_v[pl.ds(g * LANE, LANE)] = jnp.minimum(cnt, E - 1)
        pltpu.sync_copy(blk_v, blk_hbm)


def _run_slots(eidx_flat, hist):
    return pl.kernel(
        _slots_body,
        out_type=[
            jax.ShapeDtypeStruct((NPAIR,), jnp.int32),
            jax.ShapeDtypeStruct((NBPAD,), jnp.int32),
        ],
        mesh=_SC_MESH(num_cores=1),
        compiler_params=_SC_PARAMS,
        scratch_types=[
            pltpu.VMEM((CPAIR,), jnp.int32),
            pltpu.VMEM((NS, LANE), jnp.int32),
            pltpu.VMEM((CPAIR,), jnp.int32),
            pltpu.VMEM((NBPAD,), jnp.int32),
        ],
    )(eidx_flat, hist)


TPW = TOK // NW
XCH = 16


def _dispatch_body(pos_hbm, x_hbm, xs_hbm, posv, xr0, xr1,
                   rs0, rs1, ss0, ss1):
    wid = lax.axis_index("s") * NC + lax.axis_index("c")
    pbase = wid * TPW * TOP_K
    tbase = wid * TPW
    lane = lax.broadcasted_iota(jnp.int32, (LANE,), 0)
    pltpu.sync_copy(pos_hbm.at[pl.ds(pbase, TPW * TOP_K)], posv)

    xrs = (xr0, xr1)
    rsems = (rs0, rs1)
    ssems = (ss0, ss1)
    nj = TPW // XCH
    reads = [None] * nj
    scat = [None] * nj

    def fire_read(j):
        b = j & 1
        reads[j] = pltpu.async_copy(
            x_hbm.at[pl.ds(tbase + j * XCH, XCH)], xrs[b], rsems[b])

    fire_read(0)
    for j in range(nj):
        b = j & 1
        reads[j].wait()
        pair0 = (j * XCH + lane) * TOP_K
        idx_even = plsc.load_gather(posv, [pair0])
        idx_odd = plsc.load_gather(posv, [pair0 + 1])
        idx_even = jnp.minimum(jnp.maximum(idx_even, 0), P - 1)
        idx_odd = jnp.minimum(jnp.maximum(idx_odd, 0), P - 1)
        cp_e = pltpu.async_copy(xrs[b], xs_hbm.at[idx_even], ssems[b])
        cp_o = pltpu.async_copy(xrs[b], xs_hbm.at[idx_odd], ssems[b])
        scat[j] = (cp_e, cp_o)
        if j + 1 < nj:
            if j >= 1:
                scat[j - 1][0].wait()
                scat[j - 1][1].wait()
            fire_read(j + 1)
    scat[nj - 2][0].wait()
    scat[nj - 2][1].wait()
    scat[nj - 1][0].wait()
    scat[nj - 1][1].wait()


def _run_dispatch(pos, x):
    return pl.kernel(
        _dispatch_body,
        out_type=jax.ShapeDtypeStruct((P, IN_DIM), jnp.float32),
        mesh=_SC_MESH(),
        compiler_params=_SC_PARAMS,
        scratch_types=[
            pltpu.VMEM((TPW * TOP_K,), jnp.int32),
            pltpu.VMEM((XCH, IN_DIM), jnp.float32),
            pltpu.VMEM((XCH, IN_DIM), jnp.float32),
            pltpu.SemaphoreType.DMA,
            pltpu.SemaphoreType.DMA,
            pltpu.SemaphoreType.DMA,
            pltpu.SemaphoreType.DMA,
        ],
    )(pos, x)


def _mlp_kernel(be_ref, xs_ref, w1_ref, b1_ref, w2_ref, b2_ref, y_ref):
    del be_ref
    h = jnp.maximum(
        jnp.dot(xs_ref[...], w1_ref[0], preferred_element_type=jnp.float32)
        + b1_ref[0], 0.0)
    y_ref[...] = jnp.dot(h, w2_ref[0], preferred_element_type=jnp.float32) + b2_ref[0]


def _run_mlp(blk_expert, xs, eW1, eb1, eW2, eb2):
    grid_spec = pltpu.PrefetchScalarGridSpec(
        num_scalar_prefetch=1,
        grid=(NB,),
        in_specs=[
            pl.BlockSpec((BM, IN_DIM), lambda b, be: (b, 0)),
            pl.BlockSpec((1, IN_DIM, HID), lambda b, be: (be[b], 0, 0)),
            pl.BlockSpec((1, 1, HID), lambda b, be: (be[b], 0, 0)),
            pl.BlockSpec((1, HID, NUM_CLASSES), lambda b, be: (be[b], 0, 0)),
            pl.BlockSpec((1, 1, NUM_CLASSES), lambda b, be: (be[b], 0, 0)),
        ],
        out_specs=pl.BlockSpec((BM, NUM_CLASSES), lambda b, be: (b, 0)),
    )
    return pl.pallas_call(
        _mlp_kernel,
        grid_spec=grid_spec,
        out_shape=jax.ShapeDtypeStruct((P, NUM_CLASSES), jnp.float32),
    )(blk_expert, xs, eW1, eb1.reshape(E, 1, HID), eW2,
      eb2.reshape(E, 1, NUM_CLASSES))


CCH = 32


def _combine_body(pos_hbm, ew_hbm, y_hbm, out_hbm, posv, ewv,
                  idx0, idx1, yv0, yv1, ov0, ov1, gs0, gs1, ws0, ws1):
    wid = lax.axis_index("s") * NC + lax.axis_index("c")
    pbase = wid * TPW * TOP_K
    tbase = wid * TPW
    lane = lax.broadcasted_iota(jnp.int32, (LANE,), 0)
    pltpu.sync_copy(pos_hbm.at[pl.ds(pbase, TPW * TOP_K)], posv)
    pltpu.sync_copy(ew_hbm.at[pl.ds(pbase, TPW * TOP_K)], ewv)

    idxs = (idx0, idx1)
    yvs = (yv0, yv1)
    ovs = (ov0, ov1)
    gsems = (gs0, gs1)
    wsems = (ws0, ws1)
    nc = TPW // CCH
    gath = [None] * nc
    wr = [None] * nc

    def fire_gather(c):
        b = c & 1
        for j in range(CCH * TOP_K // LANE):
            v = posv[pl.ds(c * CCH * TOP_K + j * LANE, LANE)]
            idxs[b][pl.ds(j * LANE, LANE)] = jnp.minimum(
                jnp.maximum(v, 0), P - 1)
        gath[c] = pltpu.async_copy(y_hbm.at[idxs[b]], yvs[b], gsems[b])

    fire_gather(0)
    for c in range(nc):
        b = c & 1
        gath[c].wait()
        if c + 1 < nc:
            fire_gather(c + 1)
        if c >= 2:
            wr[c - 2].wait()

        def add_step(i, _):
            g0 = (i >> 3) * LANE
            ewg = ewv[pl.ds(c * CCH * TOP_K + g0, LANE)]
            l0 = (2 * i) & (LANE - 1)
            w0 = jnp.sum(jnp.where(lane == l0, ewg, 0.0))
            w1 = jnp.sum(jnp.where(lane == l0 + 1, ewg, 0.0))
            for j in range(NUM_CLASSES // LANE):
                a = yvs[b][2 * i, pl.ds(j * LANE, LANE)]
                bb = yvs[b][2 * i + 1, pl.ds(j * LANE, LANE)]
                ovs[b][i, pl.ds(j * LANE, LANE)] = a * w0 + bb * w1
            return 0

        lax.fori_loop(0, CCH, add_step, 0)
        wr[c] = pltpu.async_copy(
            ovs[b], out_hbm.at[pl.ds(tbase + c * CCH, CCH)], wsems[b])
    wr[nc - 2].wait()
    wr[nc - 1].wait()


def _run_combine(pos, ew_flat, y):
    return pl.kernel(
        _combine_body,
        out_type=jax.ShapeDtypeStruct((TOK, NUM_CLASSES), jnp.float32),
        mesh=_SC_MESH(),
        compiler_params=_SC_PARAMS,
        scratch_types=[
            pltpu.VMEM((TPW * TOP_K,), jnp.int32),
            pltpu.VMEM((TPW * TOP_K,), jnp.float32),
            pltpu.VMEM((CCH * TOP_K,), jnp.int32),
            pltpu.VMEM((CCH * TOP_K,), jnp.int32),
            pltpu.VMEM((CCH * TOP_K, NUM_CLASSES), jnp.float32),
            pltpu.VMEM((CCH * TOP_K, NUM_CLASSES), jnp.float32),
            pltpu.VMEM((CCH, NUM_CLASSES), jnp.float32),
            pltpu.VMEM((CCH, NUM_CLASSES), jnp.float32),
            pltpu.SemaphoreType.DMA,
            pltpu.SemaphoreType.DMA,
            pltpu.SemaphoreType.DMA,
            pltpu.SemaphoreType.DMA,
        ],
    )(pos, ew_flat, y)


def kernel(x, router_W1, router_b1, router_W2, router_b2,
           exp_W1, exp_b1, exp_W2, exp_b2):
    router_p, eidx, ew, hist = _run_router(
        x, router_W1, router_b1, router_W2, router_b2)

    pos, blk_expert = _run_slots(eidx.reshape(NPAIR), hist)

    xs = _run_dispatch(pos, x)
    y = _run_mlp(blk_expert, xs, exp_W1, exp_b1, exp_W2, exp_b2)
    out = _run_combine(pos, ew.reshape(NPAIR), y)

    lb_loss = jnp.asarray(0.0, jnp.float32)
    return (out, router_p, lb_loss)
